# baseline (speedup 1.0000x reference)
import functools

import jax
import jax.numpy as jnp
from jax import lax
from jax.experimental import pallas as pl
from jax.experimental.pallas import tpu as pltpu

_ROT = ((13, 15, 26, 6), (17, 29, 16, 24))
_GOLD = 466688986
_CB = 8
_ELEM = 128 * 16 * 16
_R, _C = 256, 128


def _cipher(k0, k1, x1):
    ks2 = k0 ^ k1 ^ _GOLD
    x0 = k0
    x1 = x1 + k1
    sched = ((k1, ks2, 1), (ks2, k0, 2), (k0, k1, 3), (k1, ks2, 4), (ks2, k0, 5))
    for i, (a, b, c) in enumerate(sched):
        for r in _ROT[i & 1]:
            x0 = x0 + x1
            x1 = lax.shift_left(x1, jnp.int32(r)) | lax.shift_right_logical(
                x1, jnp.int32(32 - r))
            x1 = x1 ^ x0
        x0 = x0 + a
        x1 = x1 + b + jnp.int32(c)
    return x0 ^ x1


def _stdp_body(w_ref, in_ref, yspk_ref, keys_ref, out_ref):
    pid = pl.program_id(0)
    xs = in_ref[0]
    for t in range(1, 7):
        xs = xs + in_ref[t]
    A = xs == 0
    notA = jnp.logical_not(A)
    base = (lax.broadcasted_iota(jnp.int32, (_R, _C), 0) * _C
            + lax.broadcasted_iota(jnp.int32, (_R, _C), 1))

    k = [(keys_ref[d, 0], keys_ref[d, 1]) for d in range(5)]

    for ci in range(_CB):
        o = pid * _CB + ci
        ysum = yspk_ref[0, o]
        for t in range(1, 7):
            ysum = ysum + yspk_ref[t, o]
        B = ysum == 0
        notB = jnp.logical_not(B)
        Cm = xs < ysum

        J = base + o * _ELEM
        m1 = lax.shift_right_logical(_cipher(k[0][0], k[0][1], J), jnp.int32(9))
        m2 = lax.shift_right_logical(_cipher(k[1][0], k[1][1], J), jnp.int32(9))
        m3 = lax.shift_right_logical(_cipher(k[2][0], k[2][1], J), jnp.int32(9))
        m4 = lax.shift_right_logical(_cipher(k[3][0], k[3][1], J), jnp.int32(9))
        m5 = lax.shift_right_logical(_cipher(k[4][0], k[4][1], J), jnp.int32(9))

        t_plus = jnp.where(notA & B, 65536,
                           jnp.where(notA & notB & jnp.logical_not(Cm), 655360, 0))
        t_minus = jnp.where(A & notB, 6291456,
                            jnp.where(notA & notB & Cm, 655360, 0))
        bern_plus = m1 < t_plus
        bern_minus = m2 < t_minus

        w = w_ref[ci]
        ratio = w * jnp.float32(1.0 / 7.0)
        f_minus_p = jnp.clip((1.0 - ratio) * (1.0 + ratio), 0.0, 1.0)
        f_plus_p = jnp.clip(ratio * (2.0 - ratio), 0.0, 1.0)
        inv23 = jnp.float32(1.0 / 8388608.0)
        F_minus = (m3.astype(jnp.float32) * inv23) < f_minus_p
        F_plus = (m4.astype(jnp.float32) * inv23) < f_plus_p
        umin_b = m5 < 262144
        F_plus = F_plus | umin_b
        F_minus = F_minus | umin_b

        inc = bern_plus & F_plus
        dec = bern_minus & F_minus
        wn = w + jnp.where(inc, 1.0, 0.0) - jnp.where(dec, 1.0, 0.0)
        out_ref[ci] = jnp.round(jnp.clip(wn, 0.0, 7.0))


@jax.jit
def kernel(input_spikes, output_spikes, weight):
    out_ch = weight.shape[2]
    w3 = weight.reshape(out_ch, _R, _C)
    inp = input_spikes.astype(jnp.int32).reshape(7, _R, _C)
    yspk = output_spikes.astype(jnp.int32).reshape(7, out_ch)
    keys = jax.random.key_data(jax.random.split(jax.random.key(42), 6))[1:6]
    keys = lax.bitcast_convert_type(keys, jnp.int32)

    out = pl.pallas_call(
        _stdp_body,
        grid=(out_ch // _CB,),
        in_specs=[
            pl.BlockSpec((_CB, _R, _C), lambda i: (i, 0, 0)),
            pl.BlockSpec((7, _R, _C), lambda i: (0, 0, 0)),
            pl.BlockSpec(memory_space=pltpu.SMEM),
            pl.BlockSpec(memory_space=pltpu.SMEM),
        ],
        out_specs=pl.BlockSpec((_CB, _R, _C), lambda i: (i, 0, 0)),
        out_shape=jax.ShapeDtypeStruct((out_ch, _R, _C), jnp.float32),
    )(w3, inp, yspk, keys)
    return out.reshape(weight.shape)


# speedup vs baseline: 1.4788x; 1.4788x over previous
import functools

import jax
import jax.numpy as jnp
from jax import lax
from jax.experimental import pallas as pl
from jax.experimental.pallas import tpu as pltpu

_ROT = ((13, 15, 26, 6), (17, 29, 16, 24))
_GOLD = 466688986
_CB = 8
_ELEM = 128 * 16 * 16
_R, _C = 256, 128


def _cipher(k0, k1, ks2, x1):
    x0 = k0
    x1 = x1 + k1
    sched = ((k1, ks2, 1), (ks2, k0, 2), (k0, k1, 3), (k1, ks2, 4), (ks2, k0, 5))
    for i, (a, b, c) in enumerate(sched):
        for r in _ROT[i & 1]:
            x0 = x0 + x1
            x1 = lax.shift_left(x1, jnp.int32(r)) | lax.shift_right_logical(
                x1, jnp.int32(32 - r))
            x1 = x1 ^ x0
        x0 = x0 + a
        x1 = x1 + b + jnp.int32(c)
    return x0 ^ x1


def _stdp_body(w_ref, in_ref, yspk_ref, keys_ref, out_ref):
    pid = pl.program_id(0)
    xs = in_ref[0]
    for t in range(1, 7):
        xs = xs + in_ref[t]
    A = xs == 0
    notA = jnp.logical_not(A)
    base = (lax.broadcasted_iota(jnp.int32, (_R, _C), 0) * _C
            + lax.broadcasted_iota(jnp.int32, (_R, _C), 1))

    k = [(keys_ref[d, 0], keys_ref[d, 1]) for d in range(5)]
    ks2 = [k0 ^ k1 ^ _GOLD for (k0, k1) in k]

    for ci in range(_CB):
        o = pid * _CB + ci
        ysum = yspk_ref[0, o]
        for t in range(1, 7):
            ysum = ysum + yspk_ref[t, o]
        B = ysum == 0
        Cm = xs < ysum

        is_plus = notA & (B | jnp.logical_not(Cm))

        t_hi = jnp.where(B, 0, 6291456)
        t_lo = jnp.where(B, 65536, 655360)
        thresh = jnp.where(A, t_hi, t_lo)

        J = base + o * _ELEM
        kb0 = jnp.where(is_plus, k[0][0], k[1][0])
        kb1 = jnp.where(is_plus, k[0][1], k[1][1])
        kb2 = jnp.where(is_plus, ks2[0], ks2[1])
        m_bern = lax.shift_right_logical(
            _cipher(kb0, kb1, kb2, J), jnp.int32(9))
        bern = m_bern < thresh

        kf0 = jnp.where(is_plus, k[3][0], k[2][0])
        kf1 = jnp.where(is_plus, k[3][1], k[2][1])
        kf2 = jnp.where(is_plus, ks2[3], ks2[2])
        m_f = lax.shift_right_logical(
            _cipher(kf0, kf1, kf2, J), jnp.int32(9))
        m5 = lax.shift_right_logical(
            _cipher(k[4][0], k[4][1], ks2[4], J), jnp.int32(9))

        w = w_ref[ci]
        ratio = w * jnp.float32(1.0 / 7.0)
        p_f = jnp.where(is_plus, ratio * (2.0 - ratio),
                        (1.0 - ratio) * (1.0 + ratio))
        p_f = jnp.clip(p_f, 0.0, 1.0)
        inv23 = jnp.float32(1.0 / 8388608.0)
        F = (m_f.astype(jnp.float32) * inv23) < p_f
        fire = bern & (F | (m5 < 262144))

        sign = jnp.where(is_plus, jnp.float32(1.0), jnp.float32(-1.0))
        wn = w + jnp.where(fire, sign, jnp.float32(0.0))
        out_ref[ci] = jnp.round(jnp.clip(wn, 0.0, 7.0))


@jax.jit
def kernel(input_spikes, output_spikes, weight):
    out_ch = weight.shape[2]
    w3 = weight.reshape(out_ch, _R, _C)
    inp = input_spikes.astype(jnp.int32).reshape(7, _R, _C)
    yspk = output_spikes.astype(jnp.int32).reshape(7, out_ch)
    keys = jax.random.key_data(jax.random.split(jax.random.key(42), 6))[1:6]
    keys = lax.bitcast_convert_type(keys, jnp.int32)

    out = pl.pallas_call(
        _stdp_body,
        grid=(out_ch // _CB,),
        in_specs=[
            pl.BlockSpec((_CB, _R, _C), lambda i: (i, 0, 0)),
            pl.BlockSpec((7, _R, _C), lambda i: (0, 0, 0)),
            pl.BlockSpec(memory_space=pltpu.SMEM),
            pl.BlockSpec(memory_space=pltpu.SMEM),
        ],
        out_specs=pl.BlockSpec((_CB, _R, _C), lambda i: (i, 0, 0)),
        out_shape=jax.ShapeDtypeStruct((out_ch, _R, _C), jnp.float32),
    )(w3, inp, yspk, keys)
    return out.reshape(weight.shape)


# speedup vs baseline: 1.4858x; 1.0047x over previous
import functools

import jax
import jax.numpy as jnp
from jax import lax
from jax.experimental import pallas as pl
from jax.experimental.pallas import tpu as pltpu

_ROT = ((13, 15, 26, 6), (17, 29, 16, 24))
_GOLD = 466688986
_CB = 16
_ELEM = 128 * 16 * 16
_R, _C = 256, 128


def _cipher(k0, k1, ks2, x1):
    x0 = k0
    x1 = x1 + k1
    sched = ((k1, ks2, 1), (ks2, k0, 2), (k0, k1, 3), (k1, ks2, 4), (ks2, k0, 5))
    for i, (a, b, c) in enumerate(sched):
        for r in _ROT[i & 1]:
            x0 = x0 + x1
            x1 = lax.shift_left(x1, jnp.int32(r)) | lax.shift_right_logical(
                x1, jnp.int32(32 - r))
            x1 = x1 ^ x0
        x0 = x0 + a
        x1 = x1 + b + jnp.int32(c)
    return x0 ^ x1


def _stdp_body(w_ref, in_ref, yspk_ref, keys_ref, out_ref):
    pid = pl.program_id(0)
    xs = in_ref[0]
    for t in range(1, 7):
        xs = xs + in_ref[t]
    A = xs == 0
    notA = jnp.logical_not(A)
    base = (lax.broadcasted_iota(jnp.int32, (_R, _C), 0) * _C
            + lax.broadcasted_iota(jnp.int32, (_R, _C), 1))

    k = [(keys_ref[d, 0], keys_ref[d, 1]) for d in range(5)]
    ks2 = [k0 ^ k1 ^ _GOLD for (k0, k1) in k]

    for ci in range(_CB):
        o = pid * _CB + ci
        ysum = yspk_ref[0, o]
        for t in range(1, 7):
            ysum = ysum + yspk_ref[t, o]
        B = ysum == 0
        Cm = xs < ysum

        is_plus = notA & (B | jnp.logical_not(Cm))

        t_hi = jnp.where(B, 0, 6291456)
        t_lo = jnp.where(B, 65536, 655360)
        thresh = jnp.where(A, t_hi, t_lo)

        J = base + o * _ELEM
        kb0 = jnp.where(is_plus, k[0][0], k[1][0])
        kb1 = jnp.where(is_plus, k[0][1], k[1][1])
        kb2 = jnp.where(is_plus, ks2[0], ks2[1])
        m_bern = lax.shift_right_logical(
            _cipher(kb0, kb1, kb2, J), jnp.int32(9))
        bern = m_bern < thresh

        kf0 = jnp.where(is_plus, k[3][0], k[2][0])
        kf1 = jnp.where(is_plus, k[3][1], k[2][1])
        kf2 = jnp.where(is_plus, ks2[3], ks2[2])
        m_f = lax.shift_right_logical(
            _cipher(kf0, kf1, kf2, J), jnp.int32(9))
        m5 = lax.shift_right_logical(
            _cipher(k[4][0], k[4][1], ks2[4], J), jnp.int32(9))

        w = w_ref[ci]
        ratio = w * jnp.float32(1.0 / 7.0)
        p_f = jnp.where(is_plus, ratio * (2.0 - ratio),
                        (1.0 - ratio) * (1.0 + ratio))
        p_f = jnp.clip(p_f, 0.0, 1.0)
        inv23 = jnp.float32(1.0 / 8388608.0)
        F = (m_f.astype(jnp.float32) * inv23) < p_f
        fire = bern & (F | (m5 < 262144))

        sign = jnp.where(is_plus, jnp.float32(1.0), jnp.float32(-1.0))
        wn = w + jnp.where(fire, sign, jnp.float32(0.0))
        out_ref[ci] = jnp.clip(wn, 0.0, 7.0)


@jax.jit
def kernel(input_spikes, output_spikes, weight):
    out_ch = weight.shape[2]
    w3 = weight.reshape(out_ch, _R, _C)
    inp = input_spikes.astype(jnp.int32).reshape(7, _R, _C)
    yspk = output_spikes.astype(jnp.int32).reshape(7, out_ch)
    keys = jax.random.key_data(jax.random.split(jax.random.key(42), 6))[1:6]
    keys = lax.bitcast_convert_type(keys, jnp.int32)

    out = pl.pallas_call(
        _stdp_body,
        grid=(out_ch // _CB,),
        in_specs=[
            pl.BlockSpec((_CB, _R, _C), lambda i: (i, 0, 0)),
            pl.BlockSpec((7, _R, _C), lambda i: (0, 0, 0)),
            pl.BlockSpec(memory_space=pltpu.SMEM),
            pl.BlockSpec(memory_space=pltpu.SMEM),
        ],
        out_specs=pl.BlockSpec((_CB, _R, _C), lambda i: (i, 0, 0)),
        out_shape=jax.ShapeDtypeStruct((out_ch, _R, _C), jnp.float32),
    )(w3, inp, yspk, keys)
    return out.reshape(weight.shape)


# speedup vs baseline: 1.4990x; 1.0089x over previous
import functools

import jax
import jax.numpy as jnp
from jax import lax
from jax.experimental import pallas as pl
from jax.experimental.pallas import tpu as pltpu

_ROT = ((13, 15, 26, 6), (17, 29, 16, 24))
_GOLD = 466688986
_CB = 16
_ELEM = 128 * 16 * 16
_R, _C = 256, 128


def _cipher(k0, k1, ks2, x1):
    x0 = k0
    x1 = x1 + k1
    sched = ((k1, ks2, 1), (ks2, k0, 2), (k0, k1, 3), (k1, ks2, 4), (ks2, k0, 5))
    for i, (a, b, c) in enumerate(sched):
        for r in _ROT[i & 1]:
            x0 = x0 + x1
            x1 = lax.shift_left(x1, jnp.int32(r)) | lax.shift_right_logical(
                x1, jnp.int32(32 - r))
            x1 = x1 ^ x0
        x0 = x0 + a
        x1 = x1 + b + jnp.int32(c)
    return x0 ^ x1


def _stdp_body(w_ref, in_ref, yspk_ref, keys_ref, out_ref):
    pid = pl.program_id(0)
    xs = in_ref[0]
    for t in range(1, 7):
        xs = xs + in_ref[t]
    A = xs == 0
    notA = jnp.logical_not(A)
    base = (lax.broadcasted_iota(jnp.int32, (_R, _C), 0) * _C
            + lax.broadcasted_iota(jnp.int32, (_R, _C), 1))

    k = [(keys_ref[d, 0], keys_ref[d, 1]) for d in range(5)]
    ks2 = [k0 ^ k1 ^ _GOLD for (k0, k1) in k]

    for ci in range(_CB):
        o = pid * _CB + ci
        ysum = yspk_ref[0, o]
        for t in range(1, 7):
            ysum = ysum + yspk_ref[t, o]
        B = ysum == 0
        Cm = xs < ysum

        is_plus = notA & (B | jnp.logical_not(Cm))

        t_hi = jnp.where(B, 0, 6291456)
        t_lo = jnp.where(B, 65536, 655360)
        thresh = jnp.where(A, t_hi, t_lo)

        J = base + o * _ELEM
        kb0 = jnp.where(is_plus, k[0][0], k[1][0])
        kb1 = jnp.where(is_plus, k[0][1], k[1][1])
        kb2 = jnp.where(is_plus, ks2[0], ks2[1])
        m_bern = lax.shift_right_logical(
            _cipher(kb0, kb1, kb2, J), jnp.int32(9))
        bern = m_bern < thresh

        kf0 = jnp.where(is_plus, k[3][0], k[2][0])
        kf1 = jnp.where(is_plus, k[3][1], k[2][1])
        kf2 = jnp.where(is_plus, ks2[3], ks2[2])
        m_f = lax.shift_right_logical(
            _cipher(kf0, kf1, kf2, J), jnp.int32(9))
        m5 = lax.shift_right_logical(
            _cipher(k[4][0], k[4][1], ks2[4], J), jnp.int32(9))

        w = w_ref[ci]
        ratio = w * jnp.float32(1.0 / 7.0)
        p_f = jnp.where(is_plus, ratio * (2.0 - ratio),
                        (1.0 - ratio) * (1.0 + ratio))
        p_f = jnp.clip(p_f, 0.0, 1.0)
        inv23 = jnp.float32(1.0 / 8388608.0)
        F = (m_f.astype(jnp.float32) * inv23) < p_f
        fire = bern & (F | (m5 < 262144))

        sign = jnp.where(is_plus, jnp.float32(1.0), jnp.float32(-1.0))
        wn = w + jnp.where(fire, sign, jnp.float32(0.0))
        out_ref[ci] = jnp.clip(wn, 0.0, 7.0)


@jax.jit
def kernel(input_spikes, output_spikes, weight):
    out_ch = weight.shape[2]
    w3 = weight.reshape(out_ch, _R, _C)
    inp = input_spikes.astype(jnp.int32).reshape(7, _R, _C)
    yspk = output_spikes.astype(jnp.int32).reshape(7, out_ch)
    keys = jax.random.key_data(jax.random.split(jax.random.key(42), 6))[1:6]
    keys = lax.bitcast_convert_type(keys, jnp.int32)

    out = pl.pallas_call(
        _stdp_body,
        grid=(out_ch // _CB,),
        in_specs=[
            pl.BlockSpec((_CB, _R, _C), lambda i: (i, 0, 0)),
            pl.BlockSpec((7, _R, _C), lambda i: (0, 0, 0)),
            pl.BlockSpec(memory_space=pltpu.SMEM),
            pl.BlockSpec(memory_space=pltpu.SMEM),
        ],
        out_specs=pl.BlockSpec((_CB, _R, _C), lambda i: (i, 0, 0)),
        out_shape=jax.ShapeDtypeStruct((out_ch, _R, _C), jnp.float32),
        compiler_params=pltpu.CompilerParams(
            dimension_semantics=("parallel",)),
    )(w3, inp, yspk, keys)
    return out.reshape(weight.shape)


# speedup vs baseline: 1.9330x; 1.2896x over previous
import functools

import jax
import jax.numpy as jnp
from jax import lax
from jax.experimental import pallas as pl
from jax.experimental.pallas import tpu as pltpu

_ROT = ((13, 15, 26, 6), (17, 29, 16, 24))
_GOLD = 466688986
_CB = 16
_ELEM = 128 * 16 * 16
_R, _C = 256, 128


def _cipher(k0, k1, ks2, x1):
    x0 = k0
    x1 = x1 + k1
    sched = ((k1, ks2, 1), (ks2, k0, 2), (k0, k1, 3), (k1, ks2, 4), (ks2, k0, 5))
    for i, (a, b, c) in enumerate(sched):
        for r in _ROT[i & 1]:
            x0 = x0 + x1
            x1 = lax.shift_left(x1, jnp.int32(r)) | lax.shift_right_logical(
                x1, jnp.int32(32 - r))
            x1 = x1 ^ x0
        x0 = x0 + a
        x1 = x1 + (b + jnp.int32(c))
    return x0 ^ x1


def _stdp_body(w_ref, in_ref, yspk_ref, keys_ref, out_ref):
    pid = pl.program_id(0)
    xs = in_ref[0]
    for t in range(1, 7):
        xs = xs + in_ref[t]
    A = xs == 0
    notA = jnp.logical_not(A)
    base = (lax.broadcasted_iota(jnp.int32, (_R, _C), 1) * _R
            + lax.broadcasted_iota(jnp.int32, (_R, _C), 0))

    k = [(keys_ref[d, 0], keys_ref[d, 1]) for d in range(5)]
    ks2 = [k0 ^ k1 ^ _GOLD for (k0, k1) in k]

    for ci in range(_CB):
        o = pid * _CB + ci
        ysum = yspk_ref[0, o]
        for t in range(1, 7):
            ysum = ysum + yspk_ref[t, o]
        B = ysum == 0
        Cm = xs < ysum

        is_plus = notA & (B | jnp.logical_not(Cm))

        t_hi = jnp.where(B, 0, 6291456)
        t_lo = jnp.where(B, 65536, 655360)
        thresh = jnp.where(A, t_hi, t_lo)

        J = base + o * _ELEM
        kb0 = jnp.where(is_plus, k[0][0], k[1][0])
        kb1 = jnp.where(is_plus, k[0][1], k[1][1])
        kb2 = jnp.where(is_plus, ks2[0], ks2[1])
        m_bern = lax.shift_right_logical(
            _cipher(kb0, kb1, kb2, J), jnp.int32(9))
        bern = m_bern < thresh

        kf0 = jnp.where(is_plus, k[3][0], k[2][0])
        kf1 = jnp.where(is_plus, k[3][1], k[2][1])
        kf2 = jnp.where(is_plus, ks2[3], ks2[2])
        m_f = lax.shift_right_logical(
            _cipher(kf0, kf1, kf2, J), jnp.int32(9))
        m5 = lax.shift_right_logical(
            _cipher(k[4][0], k[4][1], ks2[4], J), jnp.int32(9))

        w = w_ref[ci]
        ratio = w * jnp.float32(1.0 / 7.0)
        p_f = jnp.where(is_plus, ratio * (2.0 - ratio),
                        (1.0 - ratio) * (1.0 + ratio))
        p_f = jnp.clip(p_f, 0.0, 1.0)
        inv23 = jnp.float32(1.0 / 8388608.0)
        F = (m_f.astype(jnp.float32) * inv23) < p_f
        fire = bern & (F | (m5 < 262144))

        sign = jnp.where(is_plus, jnp.float32(1.0), jnp.float32(-1.0))
        wn = w + jnp.where(fire, sign, jnp.float32(0.0))
        out_ref[ci] = jnp.clip(wn, 0.0, 7.0)


@jax.jit
def kernel(input_spikes, output_spikes, weight):
    out_ch = weight.shape[2]
    in_ch = weight.shape[3]
    w3 = weight.reshape(out_ch, in_ch, 16, 16).transpose(0, 2, 3, 1)
    w3 = w3.reshape(out_ch, _R, _C)
    inp = input_spikes.astype(jnp.int32).transpose(0, 2, 3, 1).reshape(7, _R, _C)
    yspk = output_spikes.astype(jnp.int32).reshape(7, out_ch)
    keys = jax.random.key_data(jax.random.split(jax.random.key(42), 6))[1:6]
    keys = lax.bitcast_convert_type(keys, jnp.int32)

    out = pl.pallas_call(
        _stdp_body,
        grid=(out_ch // _CB,),
        in_specs=[
            pl.BlockSpec((_CB, _R, _C), lambda i: (i, 0, 0)),
            pl.BlockSpec((7, _R, _C), lambda i: (0, 0, 0)),
            pl.BlockSpec(memory_space=pltpu.SMEM),
            pl.BlockSpec(memory_space=pltpu.SMEM),
        ],
        out_specs=pl.BlockSpec((_CB, _R, _C), lambda i: (i, 0, 0)),
        out_shape=jax.ShapeDtypeStruct((out_ch, _R, _C), jnp.float32),
        compiler_params=pltpu.CompilerParams(
            dimension_semantics=("parallel",)),
    )(w3, inp, yspk, keys)
    out = out.reshape(out_ch, 16, 16, in_ch).transpose(0, 3, 1, 2)
    return out.reshape(weight.shape)
